# Initial kernel scaffold; baseline (speedup 1.0000x reference)
import jax
import jax.numpy as jnp
from jax.experimental import pallas as pl


def kernel(anchor, positive, negative):
    raise NotImplementedError("write your pallas kernel here")


# speedup vs baseline: 2.9762x; 2.9762x over previous
import jax
import jax.numpy as jnp
from jax.experimental import pallas as pl
from jax.experimental.pallas import tpu as pltpu


def _koleo_body(a_ref, b_ref, out_ref, rowmin_ref, gmax_ref):
    j = pl.program_id(0)
    a = a_ref[...]
    b = b_ref[...]
    s = jnp.sum(a * a, axis=1, keepdims=True)
    t = jnp.sum(b * b, axis=1)
    c = jax.lax.dot_general(
        a, b, dimension_numbers=(((1,), (1,)), ((), ())),
        preferred_element_type=jnp.float32)
    d2 = (s + t[None, :]) - 2.0 * c
    pos = jnp.where(d2 > 0.0, d2, jnp.inf)
    tmin = jnp.min(pos, axis=1, keepdims=True)
    tmax = jnp.maximum(jnp.max(d2), 0.0)

    @pl.when(j == 0)
    def _init():
        rowmin_ref[...] = tmin
        gmax_ref[0, 0] = tmax

    @pl.when(j != 0)
    def _acc():
        rowmin_ref[...] = jnp.minimum(rowmin_ref[...], tmin)
        gmax_ref[0, 0] = jnp.maximum(gmax_ref[0, 0], tmax)

    @pl.when(j == pl.num_programs(0) - 1)
    def _finish():
        rm = rowmin_ref[...]
        g = gmax_ref[0, 0]
        rm = jnp.where(rm == jnp.inf, g, rm)
        n = rm.shape[0]
        loss = -0.5 * jnp.sum(jnp.log(rm)) / n
        out_ref[...] = jnp.reshape(loss, (1, 1))


def kernel(anchor, positive, negative):
    n, d = anchor.shape
    b = jnp.concatenate([anchor, positive, negative], axis=0)
    tile = 1024
    grid = b.shape[0] // tile
    out = pl.pallas_call(
        _koleo_body,
        grid=(grid,),
        in_specs=[
            pl.BlockSpec((n, d), lambda j: (0, 0)),
            pl.BlockSpec((tile, d), lambda j: (j, 0)),
        ],
        out_specs=pl.BlockSpec((1, 1), lambda j: (0, 0)),
        out_shape=jax.ShapeDtypeStruct((1, 1), jnp.float32),
        scratch_shapes=[
            pltpu.VMEM((n, 1), jnp.float32),
            pltpu.SMEM((1, 1), jnp.float32),
        ],
        compiler_params=pltpu.CompilerParams(
            dimension_semantics=("arbitrary",)),
    )(anchor, b)
    return out[0, 0]
